# baseline (speedup 1.0000x reference)
import jax
import jax.numpy as jnp
from jax.experimental import pallas as pl
from jax.experimental.pallas import tpu as pltpu

_K = 8192
_KB = 512
_T = 1024
_WIN = 8


def _vq_kernel(a_ref, e2_ref, zb_ref, e_ref, out_ref, wv_s, wi_s, gv_s, gi_s):
    k = pl.program_id(1)
    nk = pl.num_programs(1)
    zb = zb_ref[0]
    e = e_ref[...]
    conv = jax.lax.dot_general(
        e, zb, dimension_numbers=(((1,), (0,)), ((), ())),
        preferred_element_type=jnp.float32)
    e2 = e2_ref[0]
    a = a_ref[0]
    dist = (a - conv) + e2
    m = jnp.min(dist, axis=0, keepdims=True)
    ids = jax.lax.broadcasted_iota(jnp.int32, (_KB, _T), 0)
    bidx = jnp.min(jnp.where(dist == m, ids, _K), axis=0, keepdims=True) + k * _KB

    @pl.when(k % _WIN == 0)
    def _():
        wv_s[...] = m
        wi_s[...] = bidx

    @pl.when(k % _WIN != 0)
    def _():
        better = m < wv_s[...]
        wi_s[...] = jnp.where(better, bidx, wi_s[...])
        wv_s[...] = jnp.where(better, m, wv_s[...])

    @pl.when(k % _WIN == _WIN - 1)
    def _():
        wv = wv_s[...]
        wi = wi_s[...]

        @pl.when(k == _WIN - 1)
        def _():
            gv_s[...] = wv.astype(jnp.bfloat16).astype(jnp.float32)
            gi_s[...] = wi

        @pl.when(k != _WIN - 1)
        def _():
            gv = gv_s[...]
            gi = gi_s[...]
            keep_v = gv < wv
            keep_i = keep_v | ((gv == wv) & (gi < wi))
            gv_s[...] = jnp.where(keep_v, gv, wv).astype(
                jnp.bfloat16).astype(jnp.float32)
            gi_s[...] = jnp.where(keep_i, gi, wi)

    @pl.when(k == nk - 1)
    def _():
        out_ref[...] = gi_s[...].reshape(1, 1, _T)


def kernel(z_e_x, embedding_weight):
    B, D, H, W = z_e_x.shape
    nt = H * W
    zb = (2.0 * z_e_x).astype(jnp.bfloat16).reshape(B, D, nt)
    a = jnp.sum(z_e_x * z_e_x, axis=1).reshape(B, 1, nt)
    e2 = jnp.sum(embedding_weight * embedding_weight, axis=1).reshape(
        _K // _KB, _KB, 1)

    grid = (B, _K // _KB)
    out = pl.pallas_call(
        _vq_kernel,
        grid=grid,
        in_specs=[
            pl.BlockSpec((1, 1, nt), lambda b, k: (b, 0, 0)),
            pl.BlockSpec((1, _KB, 1), lambda b, k: (k, 0, 0)),
            pl.BlockSpec((1, D, nt), lambda b, k: (b, 0, 0)),
            pl.BlockSpec((_KB, D), lambda b, k: (k, 0)),
        ],
        out_specs=pl.BlockSpec((1, 1, nt), lambda b, k: (b, 0, 0)),
        out_shape=jax.ShapeDtypeStruct((B, 1, nt), jnp.int32),
        scratch_shapes=[
            pltpu.VMEM((1, _T), jnp.float32),
            pltpu.VMEM((1, _T), jnp.int32),
            pltpu.VMEM((1, _T), jnp.float32),
            pltpu.VMEM((1, _T), jnp.int32),
        ],
        compiler_params=pltpu.CompilerParams(
            dimension_semantics=("parallel", "arbitrary")),
    )(a, e2, zb, embedding_weight)
    return out.reshape(B, H, W)


# speedup vs baseline: 1.0244x; 1.0244x over previous
import jax
import jax.numpy as jnp
from jax.experimental import pallas as pl
from jax.experimental.pallas import tpu as pltpu

_K = 8192
_KB = 512
_T = 1024
_WIN = 8


def _vq_kernel(a_ref, e2_ref, ids_ref, zb_ref, e_ref, out_ref,
               wv_s, wi_s, gv_s, gi_s):
    k = pl.program_id(1)
    nk = pl.num_programs(1)
    zb = zb_ref[0]
    e = e_ref[...]
    conv = jax.lax.dot_general(
        e, zb, dimension_numbers=(((1,), (0,)), ((), ())),
        preferred_element_type=jnp.float32)
    e2 = e2_ref[0]
    a = a_ref[0]
    dist = (a - conv) + e2
    m = jnp.min(dist, axis=0, keepdims=True)
    ids = ids_ref[...]
    bidx = jnp.min(jnp.where(dist == m, ids, float(_K)), axis=0,
                   keepdims=True).astype(jnp.int32) + k * _KB

    @pl.when(k % _WIN == 0)
    def _():
        wv_s[...] = m
        wi_s[...] = bidx

    @pl.when(k % _WIN != 0)
    def _():
        better = m < wv_s[...]
        wi_s[...] = jnp.where(better, bidx, wi_s[...])
        wv_s[...] = jnp.where(better, m, wv_s[...])

    @pl.when(k % _WIN == _WIN - 1)
    def _():
        wv = wv_s[...]
        wi = wi_s[...]

        @pl.when(k == _WIN - 1)
        def _():
            gv_s[...] = wv.astype(jnp.bfloat16).astype(jnp.float32)
            gi_s[...] = wi

        @pl.when(k != _WIN - 1)
        def _():
            gv = gv_s[...]
            gi = gi_s[...]
            keep_v = gv < wv
            keep_i = keep_v | ((gv == wv) & (gi < wi))
            gv_s[...] = jnp.where(keep_v, gv, wv).astype(
                jnp.bfloat16).astype(jnp.float32)
            gi_s[...] = jnp.where(keep_i, gi, wi)

    @pl.when(k == nk - 1)
    def _():
        out_ref[...] = gi_s[...].reshape(1, 1, _T)


def kernel(z_e_x, embedding_weight):
    B, D, H, W = z_e_x.shape
    nt = H * W
    zb = (2.0 * z_e_x).astype(jnp.bfloat16).reshape(B, D, nt)
    a = jnp.sum(z_e_x * z_e_x, axis=1).reshape(B, 1, nt)
    e2 = jnp.sum(embedding_weight * embedding_weight, axis=1).reshape(
        _K // _KB, _KB, 1)
    ids = jax.lax.broadcasted_iota(jnp.float32, (_KB, nt), 0)

    grid = (B, _K // _KB)
    out = pl.pallas_call(
        _vq_kernel,
        grid=grid,
        in_specs=[
            pl.BlockSpec((1, 1, nt), lambda b, k: (b, 0, 0)),
            pl.BlockSpec((1, _KB, 1), lambda b, k: (k, 0, 0)),
            pl.BlockSpec((_KB, nt), lambda b, k: (0, 0)),
            pl.BlockSpec((1, D, nt), lambda b, k: (b, 0, 0)),
            pl.BlockSpec((_KB, D), lambda b, k: (k, 0)),
        ],
        out_specs=pl.BlockSpec((1, 1, nt), lambda b, k: (b, 0, 0)),
        out_shape=jax.ShapeDtypeStruct((B, 1, nt), jnp.int32),
        scratch_shapes=[
            pltpu.VMEM((1, _T), jnp.float32),
            pltpu.VMEM((1, _T), jnp.int32),
            pltpu.VMEM((1, _T), jnp.float32),
            pltpu.VMEM((1, _T), jnp.int32),
        ],
        compiler_params=pltpu.CompilerParams(
            dimension_semantics=("parallel", "arbitrary")),
    )(a, e2, ids, zb, embedding_weight)
    return out.reshape(B, H, W)


# speedup vs baseline: 1.4860x; 1.4506x over previous
import jax
import jax.numpy as jnp
from jax.experimental import pallas as pl
from jax.experimental.pallas import tpu as pltpu

_K = 8192
_KB = 512
_NC = 8
_WIN = _KB * _NC
_T = 1024


def _vq_kernel(a_ref, e2_ref, ids_ref, zb_ref, e_ref, out_ref, gv_s, gi_s):
    k = pl.program_id(1)
    zb = zb_ref[0]
    a = a_ref[0]
    ids = ids_ref[...]

    win_v = None
    win_i = None
    for j in range(_NC):
        e = e_ref[j * _KB:(j + 1) * _KB, :]
        conv = jax.lax.dot_general(
            e, zb, dimension_numbers=(((1,), (0,)), ((), ())),
            preferred_element_type=jnp.float32)
        e2 = e2_ref[0, j * _KB:(j + 1) * _KB, :]
        dist = (a - conv) + e2
        m = jnp.min(dist, axis=0, keepdims=True)
        bi = (jnp.min(jnp.where(dist == m, ids, float(_K)), axis=0,
                      keepdims=True).astype(jnp.int32) + j * _KB)
        if j == 0:
            win_v, win_i = m, bi
        else:
            upd = m < win_v
            win_i = jnp.where(upd, bi, win_i)
            win_v = jnp.where(upd, m, win_v)
    win_i = win_i + k * _WIN

    @pl.when(k == 0)
    def _():
        gv_s[...] = win_v.astype(jnp.bfloat16).astype(jnp.float32)
        gi_s[...] = win_i

    @pl.when(k > 0)
    def _():
        gv = gv_s[...]
        gi = gi_s[...]
        keep_v = gv < win_v
        keep_i = keep_v | ((gv == win_v) & (gi < win_i))
        gv_s[...] = jnp.where(keep_v, gv, win_v).astype(
            jnp.bfloat16).astype(jnp.float32)
        gi_s[...] = jnp.where(keep_i, gi, win_i)

    @pl.when(k == pl.num_programs(1) - 1)
    def _():
        out_ref[...] = gi_s[...].reshape(1, 1, _T)


def kernel(z_e_x, embedding_weight):
    B, D, H, W = z_e_x.shape
    nt = H * W
    zb = (2.0 * z_e_x).astype(jnp.bfloat16).reshape(B, D, nt)
    a = jnp.sum(z_e_x * z_e_x, axis=1).reshape(B, 1, nt)
    e2 = jnp.sum(embedding_weight * embedding_weight, axis=1).reshape(
        _K // _WIN, _WIN, 1)
    ids = jax.lax.broadcasted_iota(jnp.float32, (_KB, nt), 0)

    grid = (B, _K // _WIN)
    out = pl.pallas_call(
        _vq_kernel,
        grid=grid,
        in_specs=[
            pl.BlockSpec((1, 1, nt), lambda b, k: (b, 0, 0)),
            pl.BlockSpec((1, _WIN, 1), lambda b, k: (k, 0, 0)),
            pl.BlockSpec((_KB, nt), lambda b, k: (0, 0)),
            pl.BlockSpec((1, D, nt), lambda b, k: (b, 0, 0)),
            pl.BlockSpec((_WIN, D), lambda b, k: (k, 0)),
        ],
        out_specs=pl.BlockSpec((1, 1, nt), lambda b, k: (b, 0, 0)),
        out_shape=jax.ShapeDtypeStruct((B, 1, nt), jnp.int32),
        scratch_shapes=[
            pltpu.VMEM((1, _T), jnp.float32),
            pltpu.VMEM((1, _T), jnp.int32),
        ],
        compiler_params=pltpu.CompilerParams(
            dimension_semantics=("parallel", "arbitrary")),
    )(a, e2, ids, zb, embedding_weight)
    return out.reshape(B, H, W)


# speedup vs baseline: 1.5101x; 1.0162x over previous
import jax
import jax.numpy as jnp
from jax.experimental import pallas as pl
from jax.experimental.pallas import tpu as pltpu

_K = 8192
_KB = 512
_NC = 8
_WIN = _KB * _NC
_T = 1024
_NB = 2


def _vq_kernel(a_ref, e2_ref, ids_ref, zb_ref, e_ref, out_ref, gv_s, gi_s):
    k = pl.program_id(1)
    ids = ids_ref[...]

    win_v = [None] * _NB
    win_i = [None] * _NB
    for j in range(_NC):
        e = e_ref[j * _KB:(j + 1) * _KB, :]
        e2 = e2_ref[0, j * _KB:(j + 1) * _KB, :]
        for b2 in range(_NB):
            zb = zb_ref[b2]
            conv = jax.lax.dot_general(
                e, zb, dimension_numbers=(((1,), (0,)), ((), ())),
                preferred_element_type=jnp.float32)
            dist = (a_ref[b2] - conv) + e2
            m = jnp.min(dist, axis=0, keepdims=True)
            bi = (jnp.min(jnp.where(dist == m, ids, float(_K)), axis=0,
                          keepdims=True).astype(jnp.int32) + j * _KB)
            if j == 0:
                win_v[b2], win_i[b2] = m, bi
            else:
                upd = m < win_v[b2]
                win_i[b2] = jnp.where(upd, bi, win_i[b2])
                win_v[b2] = jnp.where(upd, m, win_v[b2])

    wv = jnp.concatenate(win_v, axis=0)
    wi = jnp.concatenate(win_i, axis=0) + k * _WIN

    @pl.when(k == 0)
    def _():
        gv_s[...] = wv.astype(jnp.bfloat16).astype(jnp.float32)
        gi_s[...] = wi

    @pl.when(k > 0)
    def _():
        gv = gv_s[...]
        gi = gi_s[...]
        keep_v = gv < wv
        keep_i = keep_v | ((gv == wv) & (gi < wi))
        gv_s[...] = jnp.where(keep_v, gv, wv).astype(
            jnp.bfloat16).astype(jnp.float32)
        gi_s[...] = jnp.where(keep_i, gi, wi)

    @pl.when(k == pl.num_programs(1) - 1)
    def _():
        out_ref[...] = gi_s[...].reshape(_NB, 1, _T)


def kernel(z_e_x, embedding_weight):
    B, D, H, W = z_e_x.shape
    nt = H * W
    zb = (2.0 * z_e_x).astype(jnp.bfloat16).reshape(B, D, nt)
    a = jnp.sum(z_e_x * z_e_x, axis=1).reshape(B, 1, nt)
    e2 = jnp.sum(embedding_weight * embedding_weight, axis=1).reshape(
        _K // _WIN, _WIN, 1)
    ids = jax.lax.broadcasted_iota(jnp.float32, (_KB, nt), 0)

    grid = (B // _NB, _K // _WIN)
    out = pl.pallas_call(
        _vq_kernel,
        grid=grid,
        in_specs=[
            pl.BlockSpec((_NB, 1, nt), lambda b, k: (b, 0, 0)),
            pl.BlockSpec((1, _WIN, 1), lambda b, k: (k, 0, 0)),
            pl.BlockSpec((_KB, nt), lambda b, k: (0, 0)),
            pl.BlockSpec((_NB, D, nt), lambda b, k: (b, 0, 0)),
            pl.BlockSpec((_WIN, D), lambda b, k: (k, 0)),
        ],
        out_specs=pl.BlockSpec((_NB, 1, nt), lambda b, k: (b, 0, 0)),
        out_shape=jax.ShapeDtypeStruct((B, 1, nt), jnp.int32),
        scratch_shapes=[
            pltpu.VMEM((_NB, _T), jnp.float32),
            pltpu.VMEM((_NB, _T), jnp.int32),
        ],
        compiler_params=pltpu.CompilerParams(
            dimension_semantics=("parallel", "arbitrary")),
    )(a, e2, ids, zb, embedding_weight)
    return out.reshape(B, H, W)


# speedup vs baseline: 2.1058x; 1.3945x over previous
import jax
import jax.numpy as jnp
from jax.experimental import pallas as pl
from jax.experimental.pallas import tpu as pltpu

_K = 8192
_KB = 512
_NC = 8
_WIN = _KB * _NC
_T = 1024
_NB = 2
_SL = 8


def _vq_kernel(a_ref, e2_ref, base_ref, zb_ref, e_ref, out_ref, gv_s, gi_s):
    k = pl.program_id(1)
    base = base_ref[...]

    win_v = [None] * _NB
    win_i = [None] * _NB
    for j in range(_NC):
        e = e_ref[j * _KB:(j + 1) * _KB, :]
        for b2 in range(_NB):
            zb = zb_ref[b2]
            conv = jax.lax.dot_general(
                e, zb, dimension_numbers=(((1,), (0,)), ((), ())),
                preferred_element_type=jnp.float32)
            a = a_ref[b2]
            acc_v = None
            acc_s = None
            for s in range(_KB // _SL):
                r0 = j * _KB + s * _SL
                e2 = e2_ref[0, r0:r0 + _SL, :]
                d = (a - conv[s * _SL:(s + 1) * _SL]) + e2
                if s == 0:
                    acc_v = d
                    acc_s = jnp.zeros((_SL, _T), jnp.float32)
                else:
                    c = d < acc_v
                    acc_v = jnp.minimum(d, acc_v)
                    acc_s = jnp.where(c, float(s), acc_s)
            full_i = acc_s * float(_SL) + base
            m = jnp.min(acc_v, axis=0, keepdims=True)
            bi = (jnp.min(jnp.where(acc_v == m, full_i, float(_K)), axis=0,
                          keepdims=True).astype(jnp.int32) + j * _KB)
            if j == 0:
                win_v[b2], win_i[b2] = m, bi
            else:
                upd = m < win_v[b2]
                win_i[b2] = jnp.where(upd, bi, win_i[b2])
                win_v[b2] = jnp.where(upd, m, win_v[b2])

    wv = jnp.concatenate(win_v, axis=0)
    wi = jnp.concatenate(win_i, axis=0) + k * _WIN

    @pl.when(k == 0)
    def _():
        gv_s[...] = wv.astype(jnp.bfloat16).astype(jnp.float32)
        gi_s[...] = wi

    @pl.when(k > 0)
    def _():
        gv = gv_s[...]
        gi = gi_s[...]
        keep_v = gv < wv
        keep_i = keep_v | ((gv == wv) & (gi < wi))
        gv_s[...] = jnp.where(keep_v, gv, wv).astype(
            jnp.bfloat16).astype(jnp.float32)
        gi_s[...] = jnp.where(keep_i, gi, wi)

    @pl.when(k == pl.num_programs(1) - 1)
    def _():
        out_ref[...] = gi_s[...].reshape(_NB, 1, _T)


def kernel(z_e_x, embedding_weight):
    B, D, H, W = z_e_x.shape
    nt = H * W
    zb = (2.0 * z_e_x).astype(jnp.bfloat16).reshape(B, D, nt)
    a = jnp.sum(z_e_x * z_e_x, axis=1).reshape(B, 1, nt)
    e2 = jnp.sum(embedding_weight * embedding_weight, axis=1).reshape(
        1, _K, 1)
    base = jax.lax.broadcasted_iota(jnp.int32, (_SL, nt), 0).astype(
        jnp.float32)

    grid = (B // _NB, _K // _WIN)
    out = pl.pallas_call(
        _vq_kernel,
        grid=grid,
        in_specs=[
            pl.BlockSpec((_NB, 1, nt), lambda b, k: (b, 0, 0)),
            pl.BlockSpec((1, _K, 1), lambda b, k: (0, 0, 0)),
            pl.BlockSpec((_SL, nt), lambda b, k: (0, 0)),
            pl.BlockSpec((_NB, D, nt), lambda b, k: (b, 0, 0)),
            pl.BlockSpec((_WIN, D), lambda b, k: (k, 0)),
        ],
        out_specs=pl.BlockSpec((_NB, 1, nt), lambda b, k: (b, 0, 0)),
        out_shape=jax.ShapeDtypeStruct((B, 1, nt), jnp.int32),
        scratch_shapes=[
            pltpu.VMEM((_NB, _T), jnp.float32),
            pltpu.VMEM((_NB, _T), jnp.int32),
        ],
        compiler_params=pltpu.CompilerParams(
            dimension_semantics=("parallel", "arbitrary")),
    )(a, e2, base, zb, embedding_weight)
    return out.reshape(B, H, W)


# speedup vs baseline: 2.1171x; 1.0054x over previous
import jax
import jax.numpy as jnp
from jax.experimental import pallas as pl
from jax.experimental.pallas import tpu as pltpu

_K = 8192
_KB = 512
_NC = 8
_WIN = _KB * _NC
_T = 1024
_NB = 4
_SL = 8


def _vq_kernel(a_ref, e2_ref, base_ref, zb_ref, e_ref, out_ref, gv_s, gi_s):
    k = pl.program_id(1)
    base = base_ref[...]

    win_v = [None] * _NB
    win_i = [None] * _NB
    for j in range(_NC):
        e = e_ref[j * _KB:(j + 1) * _KB, :]
        for b2 in range(_NB):
            zb = zb_ref[b2]
            conv = jax.lax.dot_general(
                e, zb, dimension_numbers=(((1,), (0,)), ((), ())),
                preferred_element_type=jnp.float32)
            a = a_ref[b2]
            acc_v = None
            acc_s = None
            for s in range(_KB // _SL):
                r0 = j * _KB + s * _SL
                e2 = e2_ref[0, r0:r0 + _SL, :]
                d = (a - conv[s * _SL:(s + 1) * _SL]) + e2
                if s == 0:
                    acc_v = d
                    acc_s = jnp.zeros((_SL, _T), jnp.float32)
                else:
                    c = d < acc_v
                    acc_v = jnp.minimum(d, acc_v)
                    acc_s = jnp.where(c, float(s), acc_s)
            full_i = acc_s * float(_SL) + base
            m = jnp.min(acc_v, axis=0, keepdims=True)
            bi = (jnp.min(jnp.where(acc_v == m, full_i, float(_K)), axis=0,
                          keepdims=True).astype(jnp.int32) + j * _KB)
            if j == 0:
                win_v[b2], win_i[b2] = m, bi
            else:
                upd = m < win_v[b2]
                win_i[b2] = jnp.where(upd, bi, win_i[b2])
                win_v[b2] = jnp.where(upd, m, win_v[b2])

    wv = jnp.concatenate(win_v, axis=0)
    wi = jnp.concatenate(win_i, axis=0) + k * _WIN

    @pl.when(k == 0)
    def _():
        gv_s[...] = wv.astype(jnp.bfloat16).astype(jnp.float32)
        gi_s[...] = wi

    @pl.when(k > 0)
    def _():
        gv = gv_s[...]
        gi = gi_s[...]
        keep_v = gv < wv
        keep_i = keep_v | ((gv == wv) & (gi < wi))
        gv_s[...] = jnp.where(keep_v, gv, wv).astype(
            jnp.bfloat16).astype(jnp.float32)
        gi_s[...] = jnp.where(keep_i, gi, wi)

    @pl.when(k == pl.num_programs(1) - 1)
    def _():
        out_ref[...] = gi_s[...].reshape(_NB, 1, _T)


def kernel(z_e_x, embedding_weight):
    B, D, H, W = z_e_x.shape
    nt = H * W
    zb = (2.0 * z_e_x).astype(jnp.bfloat16).reshape(B, D, nt)
    a = jnp.sum(z_e_x * z_e_x, axis=1).reshape(B, 1, nt)
    e2 = jnp.sum(embedding_weight * embedding_weight, axis=1).reshape(
        1, _K, 1)
    base = jax.lax.broadcasted_iota(jnp.int32, (_SL, nt), 0).astype(
        jnp.float32)

    grid = (B // _NB, _K // _WIN)
    out = pl.pallas_call(
        _vq_kernel,
        grid=grid,
        in_specs=[
            pl.BlockSpec((_NB, 1, nt), lambda b, k: (b, 0, 0)),
            pl.BlockSpec((1, _K, 1), lambda b, k: (0, 0, 0)),
            pl.BlockSpec((_SL, nt), lambda b, k: (0, 0)),
            pl.BlockSpec((_NB, D, nt), lambda b, k: (b, 0, 0)),
            pl.BlockSpec((_WIN, D), lambda b, k: (k, 0)),
        ],
        out_specs=pl.BlockSpec((_NB, 1, nt), lambda b, k: (b, 0, 0)),
        out_shape=jax.ShapeDtypeStruct((B, 1, nt), jnp.int32),
        scratch_shapes=[
            pltpu.VMEM((_NB, _T), jnp.float32),
            pltpu.VMEM((_NB, _T), jnp.int32),
        ],
        compiler_params=pltpu.CompilerParams(
            dimension_semantics=("parallel", "arbitrary")),
    )(a, e2, base, zb, embedding_weight)
    return out.reshape(B, H, W)


# speedup vs baseline: 2.1952x; 1.0369x over previous
import jax
import jax.numpy as jnp
from jax.experimental import pallas as pl
from jax.experimental.pallas import tpu as pltpu

_K = 8192
_KB = 512
_NC = 8
_WIN = _KB * _NC
_T = 1024
_NB = 4
_SL = 8


def _vq_kernel(a_ref, e2_ref, base_ref, zb_ref, e_ref, out_ref, gv_s, gi_s):
    k = pl.program_id(1)
    base = base_ref[...]

    win_v = [None] * _NB
    win_i = [None] * _NB
    acc_v = [None] * _NB
    acc_s = [None] * _NB
    for j in range(_NC):
        e = e_ref[j * _KB:(j + 1) * _KB, :]
        for b2 in range(_NB):
            zb = zb_ref[b2]
            conv = jax.lax.dot_general(
                e, zb, dimension_numbers=(((1,), (0,)), ((), ())),
                preferred_element_type=jnp.float32)
            a = a_ref[b2]
            for s in range(_KB // _SL):
                r0 = j * _KB + s * _SL
                e2 = e2_ref[0, r0:r0 + _SL, :]
                d = (a - conv[s * _SL:(s + 1) * _SL]) + e2
                if j == 0 and s == 0:
                    acc_v[b2] = d
                    acc_s[b2] = jnp.zeros((_SL, _T), jnp.float32)
                else:
                    g = float(j * (_KB // _SL) + s)
                    c = d < acc_v[b2]
                    acc_v[b2] = jnp.minimum(d, acc_v[b2])
                    acc_s[b2] = jnp.where(c, g, acc_s[b2])

    for b2 in range(_NB):
        full_i = acc_s[b2] * float(_SL) + base
        m = jnp.min(acc_v[b2], axis=0, keepdims=True)
        win_v[b2] = m
        win_i[b2] = jnp.min(jnp.where(acc_v[b2] == m, full_i, float(_K)),
                            axis=0, keepdims=True).astype(jnp.int32)

    wv = jnp.concatenate(win_v, axis=0)
    wi = jnp.concatenate(win_i, axis=0) + k * _WIN

    @pl.when(k == 0)
    def _():
        gv_s[...] = wv.astype(jnp.bfloat16).astype(jnp.float32)
        gi_s[...] = wi

    @pl.when(k > 0)
    def _():
        gv = gv_s[...]
        gi = gi_s[...]
        keep_v = gv < wv
        keep_i = keep_v | ((gv == wv) & (gi < wi))
        gv_s[...] = jnp.where(keep_v, gv, wv).astype(
            jnp.bfloat16).astype(jnp.float32)
        gi_s[...] = jnp.where(keep_i, gi, wi)

    @pl.when(k == pl.num_programs(1) - 1)
    def _():
        out_ref[...] = gi_s[...].reshape(_NB, 1, _T)


def kernel(z_e_x, embedding_weight):
    B, D, H, W = z_e_x.shape
    nt = H * W
    zb = (2.0 * z_e_x).astype(jnp.bfloat16).reshape(B, D, nt)
    a = jnp.sum(z_e_x * z_e_x, axis=1).reshape(B, 1, nt)
    e2 = jnp.sum(embedding_weight * embedding_weight, axis=1).reshape(
        1, _K, 1)
    base = jax.lax.broadcasted_iota(jnp.int32, (_SL, nt), 0).astype(
        jnp.float32)

    grid = (B // _NB, _K // _WIN)
    out = pl.pallas_call(
        _vq_kernel,
        grid=grid,
        in_specs=[
            pl.BlockSpec((_NB, 1, nt), lambda b, k: (b, 0, 0)),
            pl.BlockSpec((1, _K, 1), lambda b, k: (0, 0, 0)),
            pl.BlockSpec((_SL, nt), lambda b, k: (0, 0)),
            pl.BlockSpec((_NB, D, nt), lambda b, k: (b, 0, 0)),
            pl.BlockSpec((_WIN, D), lambda b, k: (k, 0)),
        ],
        out_specs=pl.BlockSpec((_NB, 1, nt), lambda b, k: (b, 0, 0)),
        out_shape=jax.ShapeDtypeStruct((B, 1, nt), jnp.int32),
        scratch_shapes=[
            pltpu.VMEM((_NB, _T), jnp.float32),
            pltpu.VMEM((_NB, _T), jnp.int32),
        ],
        compiler_params=pltpu.CompilerParams(
            dimension_semantics=("parallel", "arbitrary")),
    )(a, e2, base, zb, embedding_weight)
    return out.reshape(B, H, W)
